# baseline (speedup 1.0000x reference)
import functools

import jax
import jax.numpy as jnp
from jax import lax
from jax.experimental import pallas as pl
from jax.experimental.pallas import tpu as pltpu
from jax.experimental.pallas import tpu_sc as plsc

N = 10000
E = 320000
N_PAD = 10240
E_PAD = 322560
D_IN = 128
N_CLASSES = 40


def _pad2(w, rows, cols):
    return jnp.pad(w, ((0, rows - w.shape[0]), (0, cols - w.shape[1])))


def _pad1(b, n, value=0.0):
    return jnp.pad(b, (0, n - b.shape[0]), constant_values=value)


def _node_mm_body(h_ref, wk_ref, bk_ref, wqv_ref, bqv_ref, ws_ref, k_ref,
                  qv_ref, s_ref):
    h = h_ref[...]
    k_ref[...] = jnp.dot(h, wk_ref[0], preferred_element_type=jnp.float32) + bk_ref[0]
    qv_ref[...] = jnp.dot(h, wqv_ref[0], preferred_element_type=jnp.float32) + bqv_ref[0]
    s_ref[...] = jnp.dot(h, ws_ref[...], preferred_element_type=jnp.float32)


def _node_mm(h, wk, bk, wqv2, bqv2, ws, F):
    npad, cin = h.shape
    F2 = F // 2
    BR = 2560
    R = npad // BR
    wk3 = jnp.stack([wk[:, :F2], wk[:, F2:]])
    bk3 = jnp.stack([bk[:F2].reshape(1, F2), bk[F2:].reshape(1, F2)])
    wqv3 = jnp.stack([wqv2[:, :F], wqv2[:, F:]])
    bqv3 = jnp.stack([bqv2[:F].reshape(1, F), bqv2[F:].reshape(1, F)])
    return pl.pallas_call(
        _node_mm_body,
        grid=(R, 2),
        in_specs=[
            pl.BlockSpec((BR, cin), lambda i, c: (i, 0)),
            pl.BlockSpec((1, cin, F2), lambda i, c: (c, 0, 0)),
            pl.BlockSpec((1, 1, F2), lambda i, c: (c, 0, 0)),
            pl.BlockSpec((1, cin, F), lambda i, c: (c, 0, 0)),
            pl.BlockSpec((1, 1, F), lambda i, c: (c, 0, 0)),
            pl.BlockSpec((cin, F), lambda i, c: (0, 0)),
        ],
        out_specs=[
            pl.BlockSpec((BR, F2), lambda i, c: (c * R + i, 0)),
            pl.BlockSpec((BR, F), lambda i, c: (c * R + i, 0)),
            pl.BlockSpec((BR, F), lambda i, c: (i, 0)),
        ],
        out_shape=[
            jax.ShapeDtypeStruct((2 * npad, F2), jnp.float32),
            jax.ShapeDtypeStruct((2 * npad, F), jnp.float32),
            jax.ShapeDtypeStruct((npad, F), jnp.float32),
        ],
    )(h, wk3, bk3, wqv3, bqv3, ws)


def _edge_mm_body(a_ref, w_ref, b_ref, o_ref):
    o_ref[...] = jnp.dot(a_ref[...], w_ref[...], preferred_element_type=jnp.float32) + b_ref[...]


def _edge_mm(edge_attr, we, be, F):
    e_rows, ed = edge_attr.shape
    BE = 8000
    grid = e_rows // BE
    return pl.pallas_call(
        _edge_mm_body,
        grid=(grid,),
        in_specs=[
            pl.BlockSpec((BE, ed), lambda i: (i, 0)),
            pl.BlockSpec((ed, F), lambda i: (0, 0)),
            pl.BlockSpec((1, F), lambda i: (0, 0)),
        ],
        out_specs=pl.BlockSpec((BE, F), lambda i: (i, 0)),
        out_shape=jax.ShapeDtypeStruct((e_rows, F), jnp.float32),
    )(edge_attr, we, be.reshape(1, F))


def _post_body(agg_ref, s_ref, b_ref, gamma_ref, beta_ref, o_ref):
    agg = jnp.concatenate([agg_ref[0, :N, :], agg_ref[1, :N, :]], axis=1)
    h = agg + s_ref[:N, :] + b_ref[...]
    h = jnp.where(h >= 0.0, h, 0.01 * h)
    mean = jnp.mean(h, axis=0, keepdims=True)
    var = jnp.mean((h - mean) * (h - mean), axis=0, keepdims=True)
    o_ref[:N, :] = gamma_ref[...] * (h - mean) * lax.rsqrt(var + 1e-5) + beta_ref[...]
    o_ref[N:, :] = jnp.zeros((N_PAD - N, o_ref.shape[1]), jnp.float32)


def _post(agg_pair, s, b, gamma, beta, F):
    return pl.pallas_call(
        _post_body,
        out_shape=jax.ShapeDtypeStruct((N_PAD, F), jnp.float32),
    )(agg_pair, s, b.reshape(1, F), gamma.reshape(1, F), beta.reshape(1, F))


_NC, _NS = 2, 16
_NW = _NC * _NS


_NBUF = 2


@functools.lru_cache(maxsize=None)
def _make_edge_sc(F):
    F2 = F // 2
    EPW = E_PAD // _NS
    C = 80
    NCHUNK = EPW // C
    RPT = N_PAD // _NS

    mesh = plsc.VectorSubcoreMesh(core_axis_name="c", subcore_axis_name="s",
                                  num_cores=_NC, num_subcores=_NS)

    def body(k_hbm, qv_hbm, e_hbm, src_hbm, dst_hbm, out_hbm, *scr):
        nb = _NBUF
        groups = []
        rest = list(scr)
        for _ in range(9):
            groups.append(tuple(rest[:nb]))
            rest = rest[nb:]
        sraw, draw, sgat, dgat, dsts, kdv, qvv, ev, msgv = groups
        aggsh = rest[0]
        sems = rest[1:]
        sem_i = tuple(sems[:nb])
        sem_g = tuple(sems[nb:2 * nb])
        sem_s = tuple(sems[2 * nb:3 * nb])
        msgv0 = msgv[0]
        c = lax.axis_index("c")
        s = lax.axis_index("s")
        roff = c * N_PAD
        tbase = s * EPW

        def fire_idx(b, ci):
            pltpu.async_copy(src_hbm.at[pl.ds(tbase + ci * C, C)], sraw[b],
                             sem_i[b])
            pltpu.async_copy(dst_hbm.at[pl.ds(tbase + ci * C, C)], draw[b],
                             sem_i[b])

        def wait_idx(b):
            pltpu.make_async_copy(src_hbm.at[pl.ds(0, C)], sraw[b],
                                  sem_i[b]).wait()
            pltpu.make_async_copy(dst_hbm.at[pl.ds(0, C)], draw[b],
                                  sem_i[b]).wait()

        def fill(b):
            for j in range(C // 16):
                sl = pl.ds(j * 16, 16)
                dgat[b][sl] = draw[b][sl] + roff
                sgat[b][sl] = sraw[b][sl] + roff

        def fire_gathers(b, ci):
            pltpu.async_copy(k_hbm.at[dgat[b]], kdv[b], sem_g[b])
            pltpu.async_copy(qv_hbm.at[sgat[b]], qvv[b], sem_g[b])
            ebase = jnp.minimum(tbase + ci * C, E - C)
            pltpu.async_copy(
                e_hbm.at[pl.ds(ebase, C), pl.ds(c * F2, F2)],
                ev[b], sem_g[b])

        def wait_gathers(b):
            pltpu.make_async_copy(k_hbm.at[dgat[b]], kdv[b], sem_g[b]).wait()
            pltpu.make_async_copy(qv_hbm.at[sgat[b]], qvv[b], sem_g[b]).wait()
            pltpu.make_async_copy(e_hbm.at[pl.ds(0, C), pl.ds(0, F2)],
                                  ev[b], sem_g[b]).wait()

        def wait_scatter(b):
            pltpu.make_async_copy(msgv[b], aggsh.at[dsts[b]],
                                  sem_s[b]).wait()

        for b in range(_NBUF):
            fire_idx(b, b)
        for b in range(_NBUF):
            wait_idx(b)
            fill(b)
            fire_gathers(b, b)
            fire_idx(b, b + _NBUF)

        def zero_row(r, carry):
            for j in range(F2 // 16):
                msgv0[r, pl.ds(j * 16, 16)] = jnp.zeros((16,), jnp.float32)
            return carry
        lax.fori_loop(0, C, zero_row, 0)
        for t in range(RPT // C):
            pltpu.sync_copy(msgv0, aggsh.at[pl.ds(s * RPT + t * C, C)])
        plsc.subcore_barrier()

        def super_chunk(g, carry):
            for b in range(_NBUF):
                ci = _NBUF * g + b
                wait_gathers(b)

                @pl.when(ci >= _NBUF)
                def _():
                    wait_scatter(b)

                @plsc.parallel_loop(0, C, step=1, unroll=2)
                def edge(r):
                    for j in range(F2 // 16):
                        sl = pl.ds(j * 16, 16)
                        kd = kdv[b][r, sl]
                        qj = qvv[b][r, sl]
                        vj = qvv[b][r, pl.ds(F2 + j * 16, 16)]
                        ee = ev[b][r, sl]
                        z = kd + qj + ee + ee
                        gate = 1.0 / (1.0 + jnp.exp(-z))
                        msgv[b][r, sl] = gate * (vj + ee)

                for j in range(C // 16):
                    sl = pl.ds(j * 16, 16)
                    dsts[b][sl] = dgat[b][sl] - roff
                pltpu.async_copy(msgv[b], aggsh.at[dsts[b]], sem_s[b],
                                 add=True)

                @pl.when(ci + _NBUF < NCHUNK)
                def _():
                    wait_idx(b)
                    fill(b)
                    fire_gathers(b, ci + _NBUF)

                    @pl.when(ci + 2 * _NBUF < NCHUNK)
                    def _():
                        fire_idx(b, ci + 2 * _NBUF)
            return carry
        lax.fori_loop(0, NCHUNK // _NBUF, super_chunk, 0)
        for b in range(_NBUF):
            wait_scatter(b)
        plsc.subcore_barrier()
        pltpu.sync_copy(aggsh.at[pl.ds(s * RPT, RPT)],
                        out_hbm.at[c, pl.ds(s * RPT, RPT)])

    return pl.kernel(
        body,
        out_type=jax.ShapeDtypeStruct((_NC, N_PAD, F2), jnp.float32),
        mesh=mesh,
        compiler_params=pltpu.CompilerParams(use_tc_tiling_on_sc=False),
        scratch_types=(
            [pltpu.VMEM((C,), jnp.int32) for _ in range(5 * _NBUF)]
            + [pltpu.VMEM((C, F2), jnp.float32) for _ in range(_NBUF)]
            + [pltpu.VMEM((C, F), jnp.float32) for _ in range(_NBUF)]
            + [pltpu.VMEM((C, F2), jnp.float32) for _ in range(_NBUF)]
            + [pltpu.VMEM((C, F2), jnp.float32) for _ in range(_NBUF)]
            + [pltpu.VMEM_SHARED((N_PAD, F2), jnp.float32)]
            + [pltpu.SemaphoreType.DMA for _ in range(3 * _NBUF)]
        ),
    )


def _layer(h_pad, edge_attr_pad, src, dst, p, nrm, F):
    cin = h_pad.shape[1]
    F2 = F // 2
    wk = _pad2(p["Wk"], cin, F)
    bk = _pad1(p["bk"], F)
    wq = _pad2(p["Wq"], cin, F)
    wv = _pad2(p["Wv"], cin, F)
    wqv2 = jnp.concatenate(
        [wq[:, :F2], wv[:, :F2], wq[:, F2:], wv[:, F2:]], axis=1)
    bq = _pad1(p["bq"], F)
    bv = _pad1(p["bv"], F)
    bqv2 = jnp.concatenate([bq[:F2], bv[:F2], bq[F2:], bv[F2:]])
    ws = _pad2(p["Ws"], cin, F)
    we = _pad2(p["We"], p["We"].shape[0], F)
    be = _pad1(p["be"], F)
    b = _pad1(p["b"], F)
    gamma = _pad1(nrm["gamma"], F, value=1.0)
    beta = _pad1(nrm["beta"], F)

    k_split, qv_split, s = _node_mm(h_pad, wk, bk, wqv2, bqv2, ws, F)
    e = _edge_mm(edge_attr_pad, we, be, F)
    agg_pair = _make_edge_sc(F)(k_split, qv_split, e, src, dst)
    return _post(agg_pair, s, b, gamma, beta, F)


def kernel(x, edge_index, batch, edge_attr, params):
    src = jnp.pad(edge_index[0], (0, E_PAD - E), constant_values=N_PAD - 1)
    dst = jnp.pad(edge_index[1], (0, E_PAD - E), constant_values=N_PAD - 1)
    h = jnp.pad(x, ((0, N_PAD - N), (0, 0)))
    for i, F in ((1, 128), (2, 128), (3, 128)):
        h = _layer(h, edge_attr, src, dst, params["conv%d" % i],
                   params["norm%d" % i], F)
    return h[:N, :N_CLASSES]


# speedup vs baseline: 1.4062x; 1.4062x over previous
import functools

import jax
import jax.numpy as jnp
from jax import lax
from jax.experimental import pallas as pl
from jax.experimental.pallas import tpu as pltpu
from jax.experimental.pallas import tpu_sc as plsc

N = 10000
E = 320000
N_PAD = 10240
E_PAD = 322560
D_IN = 128
N_CLASSES = 40


def _pad2(w, rows, cols):
    return jnp.pad(w, ((0, rows - w.shape[0]), (0, cols - w.shape[1])))


def _pad1(b, n, value=0.0):
    return jnp.pad(b, (0, n - b.shape[0]), constant_values=value)


def _node_mm_body(h_ref, wk_ref, bk_ref, wqv_ref, bqv_ref, ws_ref, k_ref,
                  qv_ref, s_ref):
    h = h_ref[...]
    k_ref[...] = jnp.dot(h, wk_ref[0], preferred_element_type=jnp.float32) + bk_ref[0]
    qv_ref[...] = jnp.dot(h, wqv_ref[0], preferred_element_type=jnp.float32) + bqv_ref[0]
    s_ref[...] = jnp.dot(h, ws_ref[...], preferred_element_type=jnp.float32)


def _node_mm(h, wk, bk, wqv2, bqv2, ws, F):
    npad, cin = h.shape
    F2 = F // 2
    BR = 2560
    R = npad // BR
    wk3 = jnp.stack([wk[:, :F2], wk[:, F2:]])
    bk3 = jnp.stack([bk[:F2].reshape(1, F2), bk[F2:].reshape(1, F2)])
    wqv3 = jnp.stack([wqv2[:, :F], wqv2[:, F:]])
    bqv3 = jnp.stack([bqv2[:F].reshape(1, F), bqv2[F:].reshape(1, F)])
    return pl.pallas_call(
        _node_mm_body,
        grid=(R, 2),
        in_specs=[
            pl.BlockSpec((BR, cin), lambda i, c: (i, 0)),
            pl.BlockSpec((1, cin, F2), lambda i, c: (c, 0, 0)),
            pl.BlockSpec((1, 1, F2), lambda i, c: (c, 0, 0)),
            pl.BlockSpec((1, cin, F), lambda i, c: (c, 0, 0)),
            pl.BlockSpec((1, 1, F), lambda i, c: (c, 0, 0)),
            pl.BlockSpec((cin, F), lambda i, c: (0, 0)),
        ],
        out_specs=[
            pl.BlockSpec((BR, F2), lambda i, c: (c * R + i, 0)),
            pl.BlockSpec((BR, F), lambda i, c: (c * R + i, 0)),
            pl.BlockSpec((BR, F), lambda i, c: (i, 0)),
        ],
        out_shape=[
            jax.ShapeDtypeStruct((2 * npad, F2), jnp.float32),
            jax.ShapeDtypeStruct((2 * npad, F), jnp.float32),
            jax.ShapeDtypeStruct((npad, F), jnp.float32),
        ],
    )(h, wk3, bk3, wqv3, bqv3, ws)


def _edge_mm_body(a_ref, w_ref, b_ref, o_ref):
    o_ref[...] = jnp.dot(a_ref[...], w_ref[...], preferred_element_type=jnp.float32) + b_ref[...]


def _edge_mm(edge_attr, we, be, F):
    e_rows, ed = edge_attr.shape
    BE = 8000
    grid = e_rows // BE
    return pl.pallas_call(
        _edge_mm_body,
        grid=(grid,),
        in_specs=[
            pl.BlockSpec((BE, ed), lambda i: (i, 0)),
            pl.BlockSpec((ed, F), lambda i: (0, 0)),
            pl.BlockSpec((1, F), lambda i: (0, 0)),
        ],
        out_specs=pl.BlockSpec((BE, F), lambda i: (i, 0)),
        out_shape=jax.ShapeDtypeStruct((e_rows, F), jnp.float32),
    )(edge_attr, we, be.reshape(1, F))


def _post_body(agg_ref, s_ref, b_ref, gamma_ref, beta_ref, o_ref):
    agg = jnp.concatenate([agg_ref[0, :N, :], agg_ref[1, :N, :]], axis=1)
    h = agg + s_ref[:N, :] + b_ref[...]
    h = jnp.where(h >= 0.0, h, 0.01 * h)
    mean = jnp.mean(h, axis=0, keepdims=True)
    var = jnp.mean((h - mean) * (h - mean), axis=0, keepdims=True)
    o_ref[:N, :] = gamma_ref[...] * (h - mean) * lax.rsqrt(var + 1e-5) + beta_ref[...]
    o_ref[N:, :] = jnp.zeros((N_PAD - N, o_ref.shape[1]), jnp.float32)


def _post(agg_pair, s, b, gamma, beta, F):
    return pl.pallas_call(
        _post_body,
        out_shape=jax.ShapeDtypeStruct((N_PAD, F), jnp.float32),
    )(agg_pair, s, b.reshape(1, F), gamma.reshape(1, F), beta.reshape(1, F))


_NC, _NS = 2, 16
_NW = _NC * _NS


_NBUF = 3


@functools.lru_cache(maxsize=None)
def _make_edge_sc(F):
    F2 = F // 2
    EPW = E_PAD // _NS
    C = 80
    NCHUNK = EPW // C
    RPT = N_PAD // _NS

    mesh = plsc.VectorSubcoreMesh(core_axis_name="c", subcore_axis_name="s",
                                  num_cores=_NC, num_subcores=_NS)

    def body(k_hbm, qv_hbm, e_hbm, src_hbm, dst_hbm, out_hbm, *scr):
        nb = _NBUF
        groups = []
        rest = list(scr)
        for _ in range(9):
            groups.append(tuple(rest[:nb]))
            rest = rest[nb:]
        sraw, draw, sgat, dgat, dsts, kdv, qvv, ev, msgv = groups
        aggsh = rest[0]
        sems = rest[1:]
        sem_i = tuple(sems[:nb])
        sem_g = tuple(sems[nb:2 * nb])
        sem_s = tuple(sems[2 * nb:3 * nb])
        msgv0 = msgv[0]
        c = lax.axis_index("c")
        s = lax.axis_index("s")
        roff = c * N_PAD
        tbase = s * EPW

        def fire_idx(b, ci):
            pltpu.async_copy(src_hbm.at[pl.ds(tbase + ci * C, C)], sraw[b],
                             sem_i[b])
            pltpu.async_copy(dst_hbm.at[pl.ds(tbase + ci * C, C)], draw[b],
                             sem_i[b])

        def wait_idx(b):
            pltpu.make_async_copy(src_hbm.at[pl.ds(0, C)], sraw[b],
                                  sem_i[b]).wait()
            pltpu.make_async_copy(dst_hbm.at[pl.ds(0, C)], draw[b],
                                  sem_i[b]).wait()

        def fill(b):
            for j in range(C // 16):
                sl = pl.ds(j * 16, 16)
                dgat[b][sl] = draw[b][sl] + roff
                sgat[b][sl] = sraw[b][sl] + roff

        def fire_gathers(b, ci):
            pltpu.async_copy(k_hbm.at[dgat[b]], kdv[b], sem_g[b])
            pltpu.async_copy(qv_hbm.at[sgat[b]], qvv[b], sem_g[b])
            ebase = jnp.minimum(tbase + ci * C, E - C)
            pltpu.async_copy(
                e_hbm.at[pl.ds(ebase, C), pl.ds(c * F2, F2)],
                ev[b], sem_g[b])

        def wait_gathers(b):
            pltpu.make_async_copy(k_hbm.at[dgat[b]], kdv[b], sem_g[b]).wait()
            pltpu.make_async_copy(qv_hbm.at[sgat[b]], qvv[b], sem_g[b]).wait()
            pltpu.make_async_copy(e_hbm.at[pl.ds(0, C), pl.ds(0, F2)],
                                  ev[b], sem_g[b]).wait()

        def wait_scatter(b):
            pltpu.make_async_copy(msgv[b], aggsh.at[dsts[b]],
                                  sem_s[b]).wait()

        for b in range(_NBUF):
            fire_idx(b, b)
        for b in range(_NBUF):
            wait_idx(b)
            fill(b)
            fire_gathers(b, b)
            fire_idx(b, b + _NBUF)

        def zero_row(r, carry):
            for j in range(F2 // 16):
                msgv0[r, pl.ds(j * 16, 16)] = jnp.zeros((16,), jnp.float32)
            return carry
        lax.fori_loop(0, C, zero_row, 0)
        for t in range(RPT // C):
            pltpu.sync_copy(msgv0, aggsh.at[pl.ds(s * RPT + t * C, C)])
        plsc.subcore_barrier()

        def super_chunk(g, carry):
            for b in range(_NBUF):
                ci = _NBUF * g + b
                wait_gathers(b)

                @pl.when(ci >= _NBUF)
                def _():
                    wait_scatter(b)

                @plsc.parallel_loop(0, C, step=1, unroll=2)
                def edge(r):
                    for j in range(F2 // 16):
                        sl = pl.ds(j * 16, 16)
                        kd = kdv[b][r, sl]
                        qj = qvv[b][r, sl]
                        vj = qvv[b][r, pl.ds(F2 + j * 16, 16)]
                        ee = ev[b][r, sl]
                        z = kd + qj + ee + ee
                        gate = 1.0 / (1.0 + jnp.exp(-z))
                        msgv[b][r, sl] = gate * (vj + ee)

                for j in range(C // 16):
                    sl = pl.ds(j * 16, 16)
                    dsts[b][sl] = dgat[b][sl] - roff
                pltpu.async_copy(msgv[b], aggsh.at[dsts[b]], sem_s[b],
                                 add=True)

                @pl.when(ci + _NBUF < NCHUNK)
                def _():
                    wait_idx(b)
                    fill(b)
                    fire_gathers(b, ci + _NBUF)

                    @pl.when(ci + 2 * _NBUF < NCHUNK)
                    def _():
                        fire_idx(b, ci + 2 * _NBUF)
            return carry
        lax.fori_loop(0, NCHUNK // _NBUF, super_chunk, 0)
        for b in range(_NBUF):
            wait_scatter(b)
        plsc.subcore_barrier()
        pltpu.sync_copy(aggsh.at[pl.ds(s * RPT, RPT)],
                        out_hbm.at[c, pl.ds(s * RPT, RPT)])

    return pl.kernel(
        body,
        out_type=jax.ShapeDtypeStruct((_NC, N_PAD, F2), jnp.float32),
        mesh=mesh,
        compiler_params=pltpu.CompilerParams(use_tc_tiling_on_sc=False),
        scratch_types=(
            [pltpu.VMEM((C,), jnp.int32) for _ in range(5 * _NBUF)]
            + [pltpu.VMEM((C, F2), jnp.float32) for _ in range(_NBUF)]
            + [pltpu.VMEM((C, F), jnp.float32) for _ in range(_NBUF)]
            + [pltpu.VMEM((C, F2), jnp.float32) for _ in range(_NBUF)]
            + [pltpu.VMEM((C, F2), jnp.float32) for _ in range(_NBUF)]
            + [pltpu.VMEM_SHARED((N_PAD, F2), jnp.float32)]
            + [pltpu.SemaphoreType.DMA for _ in range(3 * _NBUF)]
        ),
    )


def _layer(h_pad, edge_attr_pad, src, dst, p, nrm, F):
    cin = h_pad.shape[1]
    F2 = F // 2
    wk = _pad2(p["Wk"], cin, F)
    bk = _pad1(p["bk"], F)
    wq = _pad2(p["Wq"], cin, F)
    wv = _pad2(p["Wv"], cin, F)
    wqv2 = jnp.concatenate(
        [wq[:, :F2], wv[:, :F2], wq[:, F2:], wv[:, F2:]], axis=1)
    bq = _pad1(p["bq"], F)
    bv = _pad1(p["bv"], F)
    bqv2 = jnp.concatenate([bq[:F2], bv[:F2], bq[F2:], bv[F2:]])
    ws = _pad2(p["Ws"], cin, F)
    we = _pad2(p["We"], p["We"].shape[0], F)
    be = _pad1(p["be"], F)
    b = _pad1(p["b"], F)
    gamma = _pad1(nrm["gamma"], F, value=1.0)
    beta = _pad1(nrm["beta"], F)

    k_split, qv_split, s = _node_mm(h_pad, wk, bk, wqv2, bqv2, ws, F)
    e = _edge_mm(edge_attr_pad, we, be, F)
    agg_pair = _make_edge_sc(F)(k_split, qv_split, e, src, dst)
    return _post(agg_pair, s, b, gamma, beta, F)


def kernel(x, edge_index, batch, edge_attr, params):
    pad_rows = N + (jnp.arange(E_PAD - E, dtype=jnp.int32) % (N_PAD - N))
    src = jnp.concatenate([edge_index[0], pad_rows])
    dst = jnp.concatenate([edge_index[1], pad_rows])
    h = jnp.pad(x, ((0, N_PAD - N), (0, 0)))
    for i, F in ((1, 128), (2, 128), (3, 128)):
        h = _layer(h, edge_attr, src, dst, params["conv%d" % i],
                   params["norm%d" % i], F)
    return h[:N, :N_CLASSES]
